# Initial kernel scaffold; baseline (speedup 1.0000x reference)
import jax
import jax.numpy as jnp
from jax.experimental import pallas as pl


def kernel(input_ids, weight, gamma, beta):
    raise NotImplementedError("write your pallas kernel here")


# speedup vs baseline: 1.3860x; 1.3860x over previous
import numpy as np
import jax
import jax.numpy as jnp
from jax import lax
from jax.experimental import pallas as pl
from jax.experimental.pallas import tpu as pltpu
from jax.experimental.pallas import tpu_sc as plsc

VOCAB = 100000
DIM = 128
BASE = 512
NC, NS = 2, 16
NW = NC * NS
LANES = 16
NB = DIM // LANES
MAGIC = 12582912.0
S0, S1 = 96, 104


def _positional_table(seq_len):
    position = np.arange(BASE, dtype=np.float32)[:, None]
    div_term = np.exp(
        np.arange(0, DIM, 2, dtype=np.float32) * (-np.log(10000.0) / DIM))
    pe = np.zeros((BASE, DIM), dtype=np.float32)
    pe[:, 0::2] = np.sin(position * div_term)
    pe[:, 1::2] = np.cos(position * div_term)
    return jnp.asarray(pe[:seq_len])


def _minmax_body(w_ref, mn_ref, mx_ref):
    i = pl.program_id(0)
    bmn = jnp.min(w_ref[...])
    bmx = jnp.max(w_ref[...])

    @pl.when(i == 0)
    def _():
        mn_ref[0, 0] = bmn
        mx_ref[0, 0] = bmx

    @pl.when(i != 0)
    def _():
        mn_ref[0, 0] = jnp.minimum(mn_ref[0, 0], bmn)
        mx_ref[0, 0] = jnp.maximum(mx_ref[0, 0], bmx)


def _weight_minmax(weight):
    rows = 2000
    grid = VOCAB // rows
    mn, mx = pl.pallas_call(
        _minmax_body,
        grid=(grid,),
        in_specs=[pl.BlockSpec((rows, DIM), lambda i: (i, 0))],
        out_specs=[
            pl.BlockSpec((1, 1), lambda i: (0, 0), memory_space=pltpu.SMEM),
            pl.BlockSpec((1, 1), lambda i: (0, 0), memory_space=pltpu.SMEM),
        ],
        out_shape=[
            jax.ShapeDtypeStruct((1, 1), jnp.float32),
            jax.ShapeDtypeStruct((1, 1), jnp.float32),
        ],
    )(weight)
    return mn[0, 0], mx[0, 0]


def _rsqrt_scalar(x):
    i = lax.bitcast_convert_type(x, jnp.int32)
    i = jnp.int32(0x5F3759DF) - (i >> 1)
    y = lax.bitcast_convert_type(i, jnp.float32)
    for _ in range(3):
        y = y * (1.5 - 0.5 * x * y * y)
    return y


def _make_sc_kernel(tokens, seq):
    tok_w = tokens // NW
    ch = seq
    nchunk = tok_w // ch
    assert tok_w % ch == 0 and tokens % NW == 0 and nchunk >= 3

    mesh = plsc.VectorSubcoreMesh(core_axis_name="c", subcore_axis_name="s")

    def body(ids_hbm, w_hbm, pe_hbm, gam_hbm, bet_hbm, cst_hbm, out_hbm,
             idx0, idx1, idx2, rows0, rows1, rows2, pev, gv, bv, cv,
             gsem0, gsem1, gsem2, ssem0, ssem1, ssem2):
        idxs = (idx0, idx1, idx2)
        rows = (rows0, rows1, rows2)
        gsems = (gsem0, gsem1, gsem2)
        ssems = (ssem0, ssem1, ssem2)

        cid = lax.axis_index("c")
        sid = lax.axis_index("s")
        wid = sid * NC + cid
        tok0 = wid * tok_w

        pltpu.sync_copy(pe_hbm, pev)
        pltpu.sync_copy(gam_hbm, gv)
        pltpu.sync_copy(bet_hbm, bv)
        pltpu.sync_copy(cst_hbm, cv)

        inv = cv[pl.ds(0, LANES)]
        zp = cv[pl.ds(LANES, LANES)]
        scale = cv[pl.ds(2 * LANES, LANES)]
        zps = zp * scale

        def pe_body(r, carry):
            for j in range(NB):
                sl = pl.ds(j * LANES, LANES)
                pev[r, sl] = pev[r, sl] - zps
            return carry
        lax.fori_loop(0, seq, pe_body, 0)

        def start_gather(c, idxb, rowsb, gsem):
            t0 = tok0 + c * ch
            pltpu.sync_copy(ids_hbm.at[pl.ds(t0, ch)], idxb)
            pltpu.async_copy(w_hbm.at[idxb.at[pl.ds(0, S0)]],
                             rowsb.at[pl.ds(0, S0)], gsem)
            pltpu.async_copy(w_hbm.at[idxb.at[pl.ds(S0, S1)]],
                             rowsb.at[pl.ds(S0, S1)], gsem)

        def wait_gather(idxb, rowsb, gsem):
            pltpu.make_async_copy(w_hbm.at[idxb.at[pl.ds(0, S0)]],
                                  rowsb.at[pl.ds(0, S0)], gsem).wait()
            pltpu.make_async_copy(w_hbm.at[idxb.at[pl.ds(S0, S1)]],
                                  rowsb.at[pl.ds(S0, S1)], gsem).wait()

        def start_scatter(c, rowsb, ssem):
            t0 = tok0 + c * ch
            pltpu.async_copy(rowsb, out_hbm.at[pl.ds(t0, ch)], ssem)

        def wait_scatter(c, rowsb, ssem):
            t0 = tok0 + c * ch
            pltpu.make_async_copy(rowsb, out_hbm.at[pl.ds(t0, ch)], ssem).wait()

        def compute_chunk(rowsb):
            def row_body(r, carry):
                es = []
                for j in range(NB):
                    sl = pl.ds(j * LANES, LANES)
                    x = rowsb[r, sl]
                    y = x * inv + zp
                    q = (y + MAGIC) - MAGIC
                    es.append(q * scale + pev[r, sl])
                vs = es[0]
                vq = es[0] * es[0]
                for j in range(1, NB):
                    vs = vs + es[j]
                    vq = vq + es[j] * es[j]
                mean = jnp.sum(vs) * (1.0 / DIM)
                var = jnp.sum(vq) * (1.0 / DIM) - mean * mean
                rstd = _rsqrt_scalar(var + 1e-5)
                for j in range(NB):
                    sl = pl.ds(j * LANES, LANES)
                    a = gv[sl] * rstd
                    rowsb[r, sl] = (es[j] - mean) * a + bv[sl]
                return carry
            lax.fori_loop(0, ch, row_body, 0)

        def step(c, b):
            wait_gather(idxs[b], rows[b], gsems[b])
            compute_chunk(rows[b])
            start_scatter(c, rows[b], ssems[b])
            g = c + 2
            b2 = (b + 2) % 3
            if isinstance(g, int) and g >= nchunk:
                return

            def issue():
                start_gather(g, idxs[b2], rows[b2], gsems[b2])

            if isinstance(g, int):
                if g >= 3:
                    wait_scatter(g - 3, rows[b2], ssems[b2])
                issue()
            else:
                @pl.when(g >= 3)
                def _():
                    wait_scatter(g - 3, rows[b2], ssems[b2])
                issue()

        start_gather(0, idx0, rows0, gsem0)
        start_gather(1, idx1, rows1, gsem1)

        nsteady = (nchunk - 2) // 3
        def outer(k, carry):
            c0 = 3 * k
            step(c0, 0)
            step(c0 + 1, 1)
            step(c0 + 2, 2)
            return carry
        lax.fori_loop(0, nsteady, outer, 0)
        for c in range(nsteady * 3, nchunk):
            step(c, c % 3)
        for c in range(nchunk - 3, nchunk):
            wait_scatter(c, rows[c % 3], ssems[c % 3])

    return pl.kernel(
        body,
        out_type=jax.ShapeDtypeStruct((tokens, DIM), jnp.float32),
        mesh=mesh,
        compiler_params=pltpu.CompilerParams(needs_layout_passes=False),
        scratch_types=[
            pltpu.VMEM((ch,), jnp.int32),
            pltpu.VMEM((ch,), jnp.int32),
            pltpu.VMEM((ch,), jnp.int32),
            pltpu.VMEM((ch, DIM), jnp.float32),
            pltpu.VMEM((ch, DIM), jnp.float32),
            pltpu.VMEM((ch, DIM), jnp.float32),
            pltpu.VMEM((seq, DIM), jnp.float32),
            pltpu.VMEM((DIM,), jnp.float32),
            pltpu.VMEM((DIM,), jnp.float32),
            pltpu.VMEM((3 * LANES,), jnp.float32),
            pltpu.SemaphoreType.DMA,
            pltpu.SemaphoreType.DMA,
            pltpu.SemaphoreType.DMA,
            pltpu.SemaphoreType.DMA,
            pltpu.SemaphoreType.DMA,
            pltpu.SemaphoreType.DMA,
        ],
    )


def kernel(input_ids, weight, gamma, beta):
    batch, seq = input_ids.shape
    tokens = batch * seq
    pe = _positional_table(seq)

    wmin, wmax = _weight_minmax(weight)
    scale = (wmax - wmin) / 255.0
    zp = -128.0 - wmin / scale
    cst = jnp.concatenate([
        jnp.full((LANES,), 1.0 / scale, jnp.float32),
        jnp.full((LANES,), zp, jnp.float32),
        jnp.full((LANES,), scale, jnp.float32),
    ])

    ids_flat = input_ids.reshape(tokens).astype(jnp.int32)
    sc = _make_sc_kernel(tokens, seq)
    out = sc(ids_flat, weight, pe, gamma, beta, cst)
    return out.reshape(batch, seq, DIM)


# speedup vs baseline: 1.8718x; 1.3505x over previous
import numpy as np
import jax
import jax.numpy as jnp
from jax import lax
from jax.experimental import pallas as pl
from jax.experimental.pallas import tpu as pltpu
from jax.experimental.pallas import tpu_sc as plsc

VOCAB = 100000
DIM = 128
BASE = 512
NC, NS = 2, 16
NW = NC * NS
LANES = 16
NB = DIM // LANES
MAGIC = 12582912.0
S0, S1 = 96, 104


def _positional_table(seq_len):
    position = np.arange(BASE, dtype=np.float32)[:, None]
    div_term = np.exp(
        np.arange(0, DIM, 2, dtype=np.float32) * (-np.log(10000.0) / DIM))
    pe = np.zeros((BASE, DIM), dtype=np.float32)
    pe[:, 0::2] = np.sin(position * div_term)
    pe[:, 1::2] = np.cos(position * div_term)
    return jnp.asarray(pe[:seq_len])


def _minmax_body(w_ref, mn_ref, mx_ref):
    i = pl.program_id(0)
    bmn = jnp.min(w_ref[...])
    bmx = jnp.max(w_ref[...])

    @pl.when(i == 0)
    def _():
        mn_ref[0, 0] = bmn
        mx_ref[0, 0] = bmx

    @pl.when(i != 0)
    def _():
        mn_ref[0, 0] = jnp.minimum(mn_ref[0, 0], bmn)
        mx_ref[0, 0] = jnp.maximum(mx_ref[0, 0], bmx)


def _weight_minmax(weight):
    rows = 2000
    grid = VOCAB // rows
    mn, mx = pl.pallas_call(
        _minmax_body,
        grid=(grid,),
        in_specs=[pl.BlockSpec((rows, DIM), lambda i: (i, 0))],
        out_specs=[
            pl.BlockSpec((1, 1), lambda i: (0, 0), memory_space=pltpu.SMEM),
            pl.BlockSpec((1, 1), lambda i: (0, 0), memory_space=pltpu.SMEM),
        ],
        out_shape=[
            jax.ShapeDtypeStruct((1, 1), jnp.float32),
            jax.ShapeDtypeStruct((1, 1), jnp.float32),
        ],
    )(weight)
    return mn[0, 0], mx[0, 0]


def _rsqrt_scalar(x):
    i = lax.bitcast_convert_type(x, jnp.int32)
    i = jnp.int32(0x5F3759DF) - (i >> 1)
    y = lax.bitcast_convert_type(i, jnp.float32)
    for _ in range(3):
        y = y * (1.5 - 0.5 * x * y * y)
    return y


def _make_sc_kernel(tokens, seq):
    tok_w = tokens // NW
    ch = seq
    nchunk = tok_w // ch
    assert tok_w % ch == 0 and tokens % NW == 0 and nchunk >= 3

    mesh = plsc.VectorSubcoreMesh(core_axis_name="c", subcore_axis_name="s")

    def body(ids_hbm, w_hbm, pe_hbm, gam_hbm, bet_hbm, cst_hbm, out_hbm,
             idx0, idx1, idx2, rows0, rows1, rows2, pev, gv, bv, cv,
             gsem0, gsem1, gsem2, ssem0, ssem1, ssem2):
        idxs = (idx0, idx1, idx2)
        rows = (rows0, rows1, rows2)
        gsems = (gsem0, gsem1, gsem2)
        ssems = (ssem0, ssem1, ssem2)

        cid = lax.axis_index("c")
        sid = lax.axis_index("s")
        wid = sid * NC + cid
        tok0 = wid * tok_w

        pltpu.sync_copy(pe_hbm, pev)
        pltpu.sync_copy(gam_hbm, gv)
        pltpu.sync_copy(bet_hbm, bv)
        pltpu.sync_copy(cst_hbm, cv)

        inv = cv[pl.ds(0, LANES)]
        zp = cv[pl.ds(LANES, LANES)]
        scale = cv[pl.ds(2 * LANES, LANES)]
        zps = zp * scale

        def pe_body(r, carry):
            for j in range(NB):
                sl = pl.ds(j * LANES, LANES)
                pev[r, sl] = pev[r, sl] - zps
            return carry
        lax.fori_loop(0, seq, pe_body, 0)

        def start_gather(c, idxb, rowsb, gsem):
            t0 = tok0 + c * ch
            pltpu.sync_copy(ids_hbm.at[pl.ds(t0, ch)], idxb)
            pltpu.async_copy(w_hbm.at[idxb.at[pl.ds(0, S0)]],
                             rowsb.at[pl.ds(0, S0)], gsem)
            pltpu.async_copy(w_hbm.at[idxb.at[pl.ds(S0, S1)]],
                             rowsb.at[pl.ds(S0, S1)], gsem)

        def wait_gather(idxb, rowsb, gsem):
            pltpu.make_async_copy(w_hbm.at[idxb.at[pl.ds(0, S0)]],
                                  rowsb.at[pl.ds(0, S0)], gsem).wait()
            pltpu.make_async_copy(w_hbm.at[idxb.at[pl.ds(S0, S1)]],
                                  rowsb.at[pl.ds(S0, S1)], gsem).wait()

        def start_scatter(c, rowsb, ssem):
            t0 = tok0 + c * ch
            pltpu.async_copy(rowsb, out_hbm.at[pl.ds(t0, ch)], ssem)

        def wait_scatter(c, rowsb, ssem):
            t0 = tok0 + c * ch
            pltpu.make_async_copy(rowsb, out_hbm.at[pl.ds(t0, ch)], ssem).wait()

        RPI = 4

        def compute_chunk(rowsb):
            def row_body(it, carry):
                r0 = it * RPI
                ess, means, rstds = [], [], []
                for k in range(RPI):
                    r = r0 + k
                    es = []
                    for j in range(NB):
                        sl = pl.ds(j * LANES, LANES)
                        x = rowsb[r, sl]
                        y = x * inv + zp
                        q = (y + MAGIC) - MAGIC
                        es.append(q * scale + pev[r, sl])
                    vs = es[0]
                    vq = es[0] * es[0]
                    for j in range(1, NB):
                        vs = vs + es[j]
                        vq = vq + es[j] * es[j]
                    mean = jnp.sum(vs) * (1.0 / DIM)
                    var = jnp.sum(vq) * (1.0 / DIM) - mean * mean
                    ess.append(es)
                    means.append(mean)
                    rstds.append(_rsqrt_scalar(var + 1e-5))
                for k in range(RPI):
                    r = r0 + k
                    for j in range(NB):
                        sl = pl.ds(j * LANES, LANES)
                        a = gv[sl] * rstds[k]
                        rowsb[r, sl] = (ess[k][j] - means[k]) * a + bv[sl]
                return carry
            lax.fori_loop(0, ch // RPI, row_body, 0)

        def step(c, b):
            wait_gather(idxs[b], rows[b], gsems[b])
            compute_chunk(rows[b])
            start_scatter(c, rows[b], ssems[b])
            g = c + 2
            b2 = (b + 2) % 3
            if isinstance(g, int) and g >= nchunk:
                return

            def issue():
                start_gather(g, idxs[b2], rows[b2], gsems[b2])

            if isinstance(g, int):
                if g >= 3:
                    wait_scatter(g - 3, rows[b2], ssems[b2])
                issue()
            else:
                @pl.when(g >= 3)
                def _():
                    wait_scatter(g - 3, rows[b2], ssems[b2])
                issue()

        start_gather(0, idx0, rows0, gsem0)
        start_gather(1, idx1, rows1, gsem1)

        nsteady = (nchunk - 2) // 3
        def outer(k, carry):
            c0 = 3 * k
            step(c0, 0)
            step(c0 + 1, 1)
            step(c0 + 2, 2)
            return carry
        lax.fori_loop(0, nsteady, outer, 0)
        for c in range(nsteady * 3, nchunk):
            step(c, c % 3)
        for c in range(nchunk - 3, nchunk):
            wait_scatter(c, rows[c % 3], ssems[c % 3])

    return pl.kernel(
        body,
        out_type=jax.ShapeDtypeStruct((tokens, DIM), jnp.float32),
        mesh=mesh,
        compiler_params=pltpu.CompilerParams(needs_layout_passes=False),
        scratch_types=[
            pltpu.VMEM((ch,), jnp.int32),
            pltpu.VMEM((ch,), jnp.int32),
            pltpu.VMEM((ch,), jnp.int32),
            pltpu.VMEM((ch, DIM), jnp.float32),
            pltpu.VMEM((ch, DIM), jnp.float32),
            pltpu.VMEM((ch, DIM), jnp.float32),
            pltpu.VMEM((seq, DIM), jnp.float32),
            pltpu.VMEM((DIM,), jnp.float32),
            pltpu.VMEM((DIM,), jnp.float32),
            pltpu.VMEM((3 * LANES,), jnp.float32),
            pltpu.SemaphoreType.DMA,
            pltpu.SemaphoreType.DMA,
            pltpu.SemaphoreType.DMA,
            pltpu.SemaphoreType.DMA,
            pltpu.SemaphoreType.DMA,
            pltpu.SemaphoreType.DMA,
        ],
    )


def kernel(input_ids, weight, gamma, beta):
    batch, seq = input_ids.shape
    tokens = batch * seq
    pe = _positional_table(seq)

    wmin, wmax = _weight_minmax(weight)
    scale = (wmax - wmin) / 255.0
    zp = -128.0 - wmin / scale
    cst = jnp.concatenate([
        jnp.full((LANES,), 1.0 / scale, jnp.float32),
        jnp.full((LANES,), zp, jnp.float32),
        jnp.full((LANES,), scale, jnp.float32),
    ])

    ids_flat = input_ids.reshape(tokens).astype(jnp.int32)
    sc = _make_sc_kernel(tokens, seq)
    out = sc(ids_flat, weight, pe, gamma, beta, cst)
    return out.reshape(batch, seq, DIM)


# speedup vs baseline: 2.0255x; 1.0821x over previous
import numpy as np
import jax
import jax.numpy as jnp
from jax import lax
from jax.experimental import pallas as pl
from jax.experimental.pallas import tpu as pltpu
from jax.experimental.pallas import tpu_sc as plsc

VOCAB = 100000
DIM = 128
BASE = 512
NC, NS = 2, 16
NW = NC * NS
LANES = 16
NB = DIM // LANES
MAGIC = 12582912.0
S0, S1 = 96, 104


def _positional_table(seq_len):
    position = np.arange(BASE, dtype=np.float32)[:, None]
    div_term = np.exp(
        np.arange(0, DIM, 2, dtype=np.float32) * (-np.log(10000.0) / DIM))
    pe = np.zeros((BASE, DIM), dtype=np.float32)
    pe[:, 0::2] = np.sin(position * div_term)
    pe[:, 1::2] = np.cos(position * div_term)
    return jnp.asarray(pe[:seq_len])


def _minmax_body(w_ref, mn_ref, mx_ref):
    i = pl.program_id(0)
    bmn = jnp.min(w_ref[...])
    bmx = jnp.max(w_ref[...])

    @pl.when(i == 0)
    def _():
        mn_ref[0, 0] = bmn
        mx_ref[0, 0] = bmx

    @pl.when(i != 0)
    def _():
        mn_ref[0, 0] = jnp.minimum(mn_ref[0, 0], bmn)
        mx_ref[0, 0] = jnp.maximum(mx_ref[0, 0], bmx)


def _weight_minmax(weight):
    rows = 2000
    grid = VOCAB // rows
    mn, mx = pl.pallas_call(
        _minmax_body,
        grid=(grid,),
        in_specs=[pl.BlockSpec((rows, DIM), lambda i: (i, 0))],
        out_specs=[
            pl.BlockSpec((1, 1), lambda i: (0, 0), memory_space=pltpu.SMEM),
            pl.BlockSpec((1, 1), lambda i: (0, 0), memory_space=pltpu.SMEM),
        ],
        out_shape=[
            jax.ShapeDtypeStruct((1, 1), jnp.float32),
            jax.ShapeDtypeStruct((1, 1), jnp.float32),
        ],
    )(weight)
    return mn[0, 0], mx[0, 0]


def _rsqrt_scalar(x):
    i = lax.bitcast_convert_type(x, jnp.int32)
    i = jnp.int32(0x5F3759DF) - (i >> 1)
    y = lax.bitcast_convert_type(i, jnp.float32)
    for _ in range(3):
        y = y * (1.5 - 0.5 * x * y * y)
    return y


def _make_sc_kernel(tokens, seq):
    tok_w = tokens // NW
    ch = seq
    nchunk = tok_w // ch
    assert tok_w % ch == 0 and tokens % NW == 0 and nchunk >= 3

    mesh = plsc.VectorSubcoreMesh(core_axis_name="c", subcore_axis_name="s")

    def body(ids_hbm, w_hbm, pe_hbm, gam_hbm, bet_hbm, cst_hbm, out_hbm,
             idx0, idx1, idx2, rows0, rows1, rows2, pev, gv, bv, cv,
             gsem0, gsem1, gsem2, ssem0, ssem1, ssem2):
        idxs = (idx0, idx1, idx2)
        rows = (rows0, rows1, rows2)
        gsems = (gsem0, gsem1, gsem2)
        ssems = (ssem0, ssem1, ssem2)

        cid = lax.axis_index("c")
        sid = lax.axis_index("s")
        wid = sid * NC + cid
        tok0 = wid * tok_w

        pltpu.sync_copy(pe_hbm, pev)
        pltpu.sync_copy(gam_hbm, gv)
        pltpu.sync_copy(bet_hbm, bv)
        pltpu.sync_copy(cst_hbm, cv)

        inv = cv[pl.ds(0, LANES)]
        zp = cv[pl.ds(LANES, LANES)]
        scale = cv[pl.ds(2 * LANES, LANES)]
        zps = zp * scale

        def pe_body(r, carry):
            for j in range(NB):
                sl = pl.ds(j * LANES, LANES)
                pev[r, sl] = pev[r, sl] - zps
            return carry
        lax.fori_loop(0, seq, pe_body, 0)

        def start_gather(c, idxb, rowsb, gsem):
            t0 = tok0 + c * ch
            pltpu.sync_copy(ids_hbm.at[pl.ds(t0, ch)], idxb)
            pltpu.async_copy(w_hbm.at[idxb.at[pl.ds(0, S0)]],
                             rowsb.at[pl.ds(0, S0)], gsem)
            pltpu.async_copy(w_hbm.at[idxb.at[pl.ds(S0, S1)]],
                             rowsb.at[pl.ds(S0, S1)], gsem)

        def wait_gather(idxb, rowsb, gsem):
            pltpu.make_async_copy(w_hbm.at[idxb.at[pl.ds(0, S0)]],
                                  rowsb.at[pl.ds(0, S0)], gsem).wait()
            pltpu.make_async_copy(w_hbm.at[idxb.at[pl.ds(S0, S1)]],
                                  rowsb.at[pl.ds(S0, S1)], gsem).wait()

        def start_scatter(c, rowsb, ssem):
            t0 = tok0 + c * ch
            pltpu.async_copy(rowsb, out_hbm.at[pl.ds(t0, ch)], ssem)

        def wait_scatter(c, rowsb, ssem):
            t0 = tok0 + c * ch
            pltpu.make_async_copy(rowsb, out_hbm.at[pl.ds(t0, ch)], ssem).wait()

        RPI = 8

        def compute_chunk(rowsb):
            def row_body(it, carry):
                r0 = it * RPI
                ess, means, rstds = [], [], []
                for k in range(RPI):
                    r = r0 + k
                    es = []
                    for j in range(NB):
                        sl = pl.ds(j * LANES, LANES)
                        x = rowsb[r, sl]
                        y = x * inv + zp
                        q = (y + MAGIC) - MAGIC
                        es.append(q * scale + pev[r, sl])
                    vs = es[0]
                    vq = es[0] * es[0]
                    for j in range(1, NB):
                        vs = vs + es[j]
                        vq = vq + es[j] * es[j]
                    mean = jnp.sum(vs) * (1.0 / DIM)
                    var = jnp.sum(vq) * (1.0 / DIM) - mean * mean
                    ess.append(es)
                    means.append(mean)
                    rstds.append(_rsqrt_scalar(var + 1e-5))
                for k in range(RPI):
                    r = r0 + k
                    for j in range(NB):
                        sl = pl.ds(j * LANES, LANES)
                        a = gv[sl] * rstds[k]
                        rowsb[r, sl] = (ess[k][j] - means[k]) * a + bv[sl]
                return carry
            lax.fori_loop(0, ch // RPI, row_body, 0)

        def step(c, b):
            wait_gather(idxs[b], rows[b], gsems[b])
            compute_chunk(rows[b])
            start_scatter(c, rows[b], ssems[b])
            g = c + 2
            b2 = (b + 2) % 3
            if isinstance(g, int) and g >= nchunk:
                return

            def issue():
                start_gather(g, idxs[b2], rows[b2], gsems[b2])

            if isinstance(g, int):
                if g >= 3:
                    wait_scatter(g - 3, rows[b2], ssems[b2])
                issue()
            else:
                @pl.when(g >= 3)
                def _():
                    wait_scatter(g - 3, rows[b2], ssems[b2])
                issue()

        start_gather(0, idx0, rows0, gsem0)
        start_gather(1, idx1, rows1, gsem1)

        nsteady = (nchunk - 2) // 3
        def outer(k, carry):
            c0 = 3 * k
            step(c0, 0)
            step(c0 + 1, 1)
            step(c0 + 2, 2)
            return carry
        lax.fori_loop(0, nsteady, outer, 0)
        for c in range(nsteady * 3, nchunk):
            step(c, c % 3)
        for c in range(nchunk - 3, nchunk):
            wait_scatter(c, rows[c % 3], ssems[c % 3])

    return pl.kernel(
        body,
        out_type=jax.ShapeDtypeStruct((tokens, DIM), jnp.float32),
        mesh=mesh,
        compiler_params=pltpu.CompilerParams(needs_layout_passes=False),
        scratch_types=[
            pltpu.VMEM((ch,), jnp.int32),
            pltpu.VMEM((ch,), jnp.int32),
            pltpu.VMEM((ch,), jnp.int32),
            pltpu.VMEM((ch, DIM), jnp.float32),
            pltpu.VMEM((ch, DIM), jnp.float32),
            pltpu.VMEM((ch, DIM), jnp.float32),
            pltpu.VMEM((seq, DIM), jnp.float32),
            pltpu.VMEM((DIM,), jnp.float32),
            pltpu.VMEM((DIM,), jnp.float32),
            pltpu.VMEM((3 * LANES,), jnp.float32),
            pltpu.SemaphoreType.DMA,
            pltpu.SemaphoreType.DMA,
            pltpu.SemaphoreType.DMA,
            pltpu.SemaphoreType.DMA,
            pltpu.SemaphoreType.DMA,
            pltpu.SemaphoreType.DMA,
        ],
    )


def kernel(input_ids, weight, gamma, beta):
    batch, seq = input_ids.shape
    tokens = batch * seq
    pe = _positional_table(seq)

    wmin, wmax = _weight_minmax(weight)
    scale = (wmax - wmin) / 255.0
    zp = -128.0 - wmin / scale
    cst = jnp.concatenate([
        jnp.full((LANES,), 1.0 / scale, jnp.float32),
        jnp.full((LANES,), zp, jnp.float32),
        jnp.full((LANES,), scale, jnp.float32),
    ])

    ids_flat = input_ids.reshape(tokens).astype(jnp.int32)
    sc = _make_sc_kernel(tokens, seq)
    out = sc(ids_flat, weight, pe, gamma, beta, cst)
    return out.reshape(batch, seq, DIM)
